# baseline (speedup 1.0000x reference)
import functools

import jax
import jax.numpy as jnp
from jax import lax
from jax.experimental import pallas as pl
from jax.experimental.pallas import tpu as pltpu
from jax.experimental.pallas import tpu_sc as plsc

K_Q = 65536
D = 128
B_K = 4096
MASK = K_Q - 1
NC, NS = 2, 16
NW = NC * NS
SLAB = K_Q // NW
PC = 128


def _sc_body(keys_hbm, queue_hbm, ptr_hbm, out_hbm, ptr_v, idx_v, buf_v, sem):
    w = lax.axis_index("s") * NC + lax.axis_index("c")
    t0 = pl.multiple_of(w * SLAB, SLAB)
    pltpu.sync_copy(ptr_hbm, ptr_v.at[pl.ds(0, 1)])
    p = ptr_v[...][0]
    d0 = (p + t0) & MASK
    aligned = (p & 7) == 0
    fast = jnp.logical_and(aligned, d0 <= K_Q - SLAB)
    is_keys = t0 < B_K

    @pl.when(jnp.logical_and(fast, is_keys))
    def _():
        d0a = pl.multiple_of(d0, 8)
        pltpu.sync_copy(keys_hbm.at[pl.ds(t0, SLAB)],
                        out_hbm.at[pl.ds(d0a, SLAB)])

    @pl.when(jnp.logical_and(fast, jnp.logical_not(is_keys)))
    def _():
        d0a = pl.multiple_of(d0, 8)
        pltpu.sync_copy(queue_hbm.at[pl.ds(d0a, SLAB)],
                        out_hbm.at[pl.ds(d0a, SLAB)])

    @pl.when(jnp.logical_not(fast))
    def _():
        for j in range(SLAB // PC):
            tp = pl.multiple_of(t0 + j * PC, PC)
            dp = (p + tp) & MASK
            lin = jnp.logical_and(aligned, dp <= K_Q - PC)
            kp = tp < B_K

            @pl.when(jnp.logical_and(lin, kp))
            def _():
                dpa = pl.multiple_of(dp, 8)
                pltpu.sync_copy(keys_hbm.at[pl.ds(tp, PC)],
                                out_hbm.at[pl.ds(dpa, PC)])

            @pl.when(jnp.logical_and(lin, jnp.logical_not(kp)))
            def _():
                dpa = pl.multiple_of(dp, 8)
                pltpu.sync_copy(queue_hbm.at[pl.ds(dpa, PC)],
                                out_hbm.at[pl.ds(dpa, PC)])

            @pl.when(jnp.logical_not(lin))
            def _():
                for q in range(PC // 16):
                    idx_v[pl.ds(q * 16, 16)] = (
                        dp + q * 16 + lax.iota(jnp.int32, 16)) & MASK

                @pl.when(kp)
                def _():
                    pltpu.sync_copy(keys_hbm.at[pl.ds(tp, PC)], buf_v)

                @pl.when(jnp.logical_not(kp))
                def _():
                    pltpu.async_copy(queue_hbm.at[idx_v], buf_v, sem).wait()

                pltpu.async_copy(buf_v, out_hbm.at[idx_v], sem).wait()


def kernel(keys, queue, ptr):
    mesh = plsc.VectorSubcoreMesh(core_axis_name="c", subcore_axis_name="s")
    run = functools.partial(
        pl.kernel,
        out_type=jax.ShapeDtypeStruct((K_Q, D), jnp.float32),
        mesh=mesh,
        scratch_types=[
            pltpu.VMEM((16,), jnp.int32),
            pltpu.VMEM((PC,), jnp.int32),
            pltpu.VMEM((PC, D), jnp.float32),
            pltpu.SemaphoreType.DMA,
        ],
    )(_sc_body)
    return run(keys, queue, ptr.astype(jnp.int32))


# speedup vs baseline: 23.0599x; 23.0599x over previous
import functools

import jax
import jax.numpy as jnp
from jax import lax
from jax.experimental import pallas as pl
from jax.experimental.pallas import tpu as pltpu
from jax.experimental.pallas import tpu_sc as plsc

K_Q = 65536
D = 128
B_K = 4096
MASK = K_Q - 1
NC, NS = 2, 16
NW = NC * NS
SLAB = K_Q // NW
CH = 256
NCH = SLAB // CH
PC = 128


def _fast_slab(src_hbm, out_hbm, s0, d0, bufs, gsems, wsems):
    gh = [None] * NCH
    wh = [None] * NCH
    gh[0] = pltpu.async_copy(src_hbm.at[pl.ds(s0, CH)], bufs[0], gsems[0])
    gh[1] = pltpu.async_copy(src_hbm.at[pl.ds(s0 + CH, CH)], bufs[1],
                             gsems[1])
    for i in range(NCH):
        b = i % 2
        gh[i].wait()
        wh[i] = pltpu.async_copy(bufs[b], out_hbm.at[pl.ds(d0 + i * CH, CH)],
                                 wsems[b])
        if i + 2 < NCH:
            wh[i].wait()
            gh[i + 2] = pltpu.async_copy(
                src_hbm.at[pl.ds(s0 + (i + 2) * CH, CH)], bufs[b], gsems[b])
    wh[NCH - 2].wait()
    wh[NCH - 1].wait()


def _sc_body(keys_hbm, queue_hbm, ptr_hbm, out_hbm,
             ptr_v, idx_v, buf_v, buf_a, buf_b, sga, sgb, swa, swb):
    w = lax.axis_index("s") * NC + lax.axis_index("c")
    t0 = pl.multiple_of(w * SLAB, SLAB)
    pltpu.sync_copy(ptr_hbm, ptr_v.at[pl.ds(0, 1)])
    p = ptr_v[...][0]
    d0 = (p + t0) & MASK
    aligned = (p & 7) == 0
    fast = jnp.logical_and(aligned, d0 <= K_Q - SLAB)
    is_keys = t0 < B_K
    bufs, gsems, wsems = (buf_a, buf_b), (sga, sgb), (swa, swb)

    @pl.when(jnp.logical_and(fast, is_keys))
    def _():
        d0a = pl.multiple_of(d0, 8)
        _fast_slab(keys_hbm, out_hbm, t0, d0a, bufs, gsems, wsems)

    @pl.when(jnp.logical_and(fast, jnp.logical_not(is_keys)))
    def _():
        d0a = pl.multiple_of(d0, 8)
        _fast_slab(queue_hbm, out_hbm, d0a, d0a, bufs, gsems, wsems)

    @pl.when(jnp.logical_not(fast))
    def _():
        for j in range(SLAB // PC):
            tp = pl.multiple_of(t0 + j * PC, PC)
            dp = (p + tp) & MASK
            lin = jnp.logical_and(aligned, dp <= K_Q - PC)
            kp = tp < B_K

            @pl.when(jnp.logical_and(lin, kp))
            def _():
                dpa = pl.multiple_of(dp, 8)
                pltpu.sync_copy(keys_hbm.at[pl.ds(tp, PC)], buf_v)
                pltpu.sync_copy(buf_v, out_hbm.at[pl.ds(dpa, PC)])

            @pl.when(jnp.logical_and(lin, jnp.logical_not(kp)))
            def _():
                dpa = pl.multiple_of(dp, 8)
                pltpu.sync_copy(queue_hbm.at[pl.ds(dpa, PC)], buf_v)
                pltpu.sync_copy(buf_v, out_hbm.at[pl.ds(dpa, PC)])

            @pl.when(jnp.logical_not(lin))
            def _():
                for q in range(PC // 16):
                    idx_v[pl.ds(q * 16, 16)] = (
                        dp + q * 16 + lax.iota(jnp.int32, 16)) & MASK

                @pl.when(kp)
                def _():
                    pltpu.sync_copy(keys_hbm.at[pl.ds(tp, PC)], buf_v)

                @pl.when(jnp.logical_not(kp))
                def _():
                    pltpu.async_copy(queue_hbm.at[idx_v], buf_v, sga).wait()

                pltpu.async_copy(buf_v, out_hbm.at[idx_v], sga).wait()


def kernel(keys, queue, ptr):
    mesh = plsc.VectorSubcoreMesh(core_axis_name="c", subcore_axis_name="s")
    run = functools.partial(
        pl.kernel,
        out_type=jax.ShapeDtypeStruct((K_Q, D), jnp.float32),
        mesh=mesh,
        scratch_types=[
            pltpu.VMEM((16,), jnp.int32),
            pltpu.VMEM((PC,), jnp.int32),
            pltpu.VMEM((PC, D), jnp.float32),
            pltpu.VMEM((CH, D), jnp.float32),
            pltpu.VMEM((CH, D), jnp.float32),
            pltpu.SemaphoreType.DMA,
            pltpu.SemaphoreType.DMA,
            pltpu.SemaphoreType.DMA,
            pltpu.SemaphoreType.DMA,
        ],
    )(_sc_body)
    return run(keys, queue, ptr.astype(jnp.int32))
